# baseline (speedup 1.0000x reference)
import jax
import jax.numpy as jnp
from jax.experimental import pallas as pl
import jax.experimental.pallas.tpu as pltpu
from jax.experimental.pallas import tpu_sc as plsc

E = 8
H = 1024
MOE_INTER = 2048
MI2 = MOE_INTER // 2
SH_INTER = 4096
T = 2048
TM = 512
TMS = 512
BLK = 512
NB = 16
NS = NB * BLK
W_SC = 128
HC = 4


def _shift_down(v, d):
    return jnp.concatenate([jnp.zeros((d, v.shape[1]), v.dtype), v[:-d, :]], axis=0)


def _shift_right(v, d):
    return jnp.concatenate([jnp.zeros((v.shape[0], d), v.dtype), v[:, :-d]], axis=1)


def _router_body(x_ref, gw_ref, sg_ref, logits_ref, xb_ref, pos_ref, w0_ref,
                 w1_ref, gate_ref, meta_ref):
    x = x_ref[...]
    xb_ref[...] = x.astype(jnp.bfloat16)
    lg = jax.lax.dot_general(x, gw_ref[...], (((1,), (1,)), ((), ())),
                             preferred_element_type=jnp.float32)
    logits_ref[...] = lg
    p = jax.nn.softmax(lg, axis=-1)
    iota = jax.lax.broadcasted_iota(jnp.int32, p.shape, 1)
    m1 = jnp.max(p, axis=-1, keepdims=True)
    pos1 = jnp.min(jnp.where(p == m1, iota, E), axis=-1, keepdims=True)
    first1 = (iota == pos1).astype(jnp.float32)
    p2 = jnp.where(first1 > 0, -jnp.inf, p)
    m2 = jnp.max(p2, axis=-1, keepdims=True)
    pos2 = jnp.min(jnp.where(p2 == m2, iota, E), axis=-1, keepdims=True)
    first2 = (iota == pos2).astype(jnp.float32)
    w0_ref[...] = m1
    w1_ref[...] = m2
    g = jnp.sum(x * sg_ref[...], axis=-1, keepdims=True)
    gate_ref[...] = jax.nn.sigmoid(g)

    onehot = first1 + first2
    incl = onehot
    d = 1
    while d < T:
        incl = incl + _shift_down(incl, d)
        d *= 2
    excl = incl - onehot
    rank0 = jnp.sum(excl * first1, axis=1, keepdims=True)
    rank1 = jnp.sum(excl * first2, axis=1, keepdims=True)
    cnt = incl[T - 1:T, :]
    pcnt = jnp.ceil(cnt * (1.0 / BLK)) * BLK
    incl8 = pcnt
    d = 1
    while d < E:
        incl8 = incl8 + _shift_right(incl8, d)
        d *= 2
    off = incl8 - pcnt
    slot0 = jnp.sum(off * first1, axis=1, keepdims=True) + rank0
    slot1 = jnp.sum(off * first2, axis=1, keepdims=True) + rank1
    slot01 = jnp.concatenate([slot0, slot1], axis=0)
    pos_ref[...] = slot01.astype(jnp.int32).T

    iota8 = jax.lax.broadcasted_iota(jnp.int32, (1, E), 1).astype(jnp.float32)
    last_e = jnp.max(jnp.where(pcnt > 0, iota8, -1.0))
    biota = jax.lax.broadcasted_iota(jnp.int32, (NB, E), 0).astype(jnp.float32)
    pincl_b = jnp.broadcast_to(incl8, (NB, E))
    ebs = jnp.sum((biota * BLK >= pincl_b).astype(jnp.float32), axis=1,
                  keepdims=True)
    ebs = jnp.minimum(ebs, last_e)
    total = incl8[0:1, E - 1:E]
    bcol = jax.lax.broadcasted_iota(jnp.int32, (NB, 1), 0).astype(jnp.float32)
    act = (bcol * BLK < total).astype(jnp.float32)
    meta_ref[...] = jnp.concatenate([ebs, act], axis=1).astype(jnp.int32)


def _grouped_body(s_ref, xs_ref, w1_ref, w3_ref, w2_ref, ys_ref, acc_ref):
    b = pl.program_id(0)
    m = pl.program_id(1)

    @pl.when(s_ref[b, 1] == 1)
    def _():
        xw = xs_ref[...].astype(jnp.bfloat16)
        w1 = w1_ref[0].astype(jnp.bfloat16)
        w3 = w3_ref[0].astype(jnp.bfloat16)
        w2 = w2_ref[0].astype(jnp.bfloat16)
        a = jax.lax.dot_general(xw, w1, (((1,), (1,)), ((), ())),
                                preferred_element_type=jnp.float32)
        c = jax.lax.dot_general(xw, w3, (((1,), (1,)), ((), ())),
                                preferred_element_type=jnp.float32)
        hm = (a * jax.nn.sigmoid(a) * c).astype(jnp.bfloat16)
        yc = jax.lax.dot_general(hm, w2, (((1,), (1,)), ((), ())),
                                 preferred_element_type=jnp.float32)

        @pl.when(m == 0)
        def _():
            acc_ref[...] = yc

        @pl.when(m != 0)
        def _():
            ys_ref[...] = acc_ref[...] + yc


def _shared_body(xb_ref, s1_ref, s3_ref, s2_ref, gate_ref, y0_ref, y1_ref,
                 w0_ref, w1_ref, o_ref, acc_ref):
    m = pl.program_id(0)
    t = pl.program_id(1)
    xw = xb_ref[...]
    s1 = s1_ref[...].astype(jnp.bfloat16)
    s3 = s3_ref[...].astype(jnp.bfloat16)
    s2 = s2_ref[...].astype(jnp.bfloat16)
    a = jax.lax.dot_general(xw, s1, (((1,), (1,)), ((), ())),
                            preferred_element_type=jnp.float32)
    c = jax.lax.dot_general(xw, s3, (((1,), (1,)), ((), ())),
                            preferred_element_type=jnp.float32)
    hm = (a * jax.nn.sigmoid(a) * c).astype(jnp.bfloat16)
    yc = jax.lax.dot_general(hm, s2, (((1,), (1,)), ((), ())),
                             preferred_element_type=jnp.float32)
    sl = pl.ds(t * TMS, TMS)

    @pl.when(m == 0)
    def _():
        acc_ref[sl, :] = yc

    @pl.when(m != 0)
    def _():
        acc_ref[sl, :] += yc

    o_ref[...] = (gate_ref[...] * acc_ref[sl, :]
                  + w0_ref[...] * y0_ref[...] + w1_ref[...] * y1_ref[...])


def _sc_scatter(x_f32, pos_sc):
    vector_mesh = plsc.VectorSubcoreMesh(core_axis_name="c",
                                         subcore_axis_name="s")

    @pl.kernel(out_type=jax.ShapeDtypeStruct((NS, H), jnp.float32),
               mesh=vector_mesh)
    def k(x_hbm, i_hbm, o_hbm):
        for hh in range(HC):
            def body(x_vmem, i_vmem, _h=hh):
                pltpu.sync_copy(
                    x_vmem, o_hbm.at[i_vmem.at[0], pl.ds(_h * (H // HC), H // HC)])

            pltpu.emit_pipeline(
                body,
                grid=(2 * T // W_SC,),
                in_specs=[
                    pl.BlockSpec((W_SC, H // HC),
                                 lambda j, _h=hh: (j % (T // W_SC), _h)),
                    pl.BlockSpec((1, W_SC), lambda j: (0, j)),
                ],
                out_specs=[],
                core_axis_name=("c", "s"),
                dimension_semantics=(pltpu.PARALLEL,),
            )(x_hbm, i_hbm)

    return k(x_f32, pos_sc)


def _sc_gather(ys, pos_sc):
    vector_mesh = plsc.VectorSubcoreMesh(core_axis_name="c",
                                         subcore_axis_name="s")

    @pl.kernel(out_type=jax.ShapeDtypeStruct((2 * T, H), jnp.float32),
               mesh=vector_mesh)
    def k(y_hbm, i_hbm, o_hbm):
        for hh in range(HC):
            def body(i_vmem, o_vmem, _h=hh):
                pltpu.sync_copy(
                    y_hbm.at[i_vmem.at[0], pl.ds(_h * (H // HC), H // HC)], o_vmem)

            pltpu.emit_pipeline(
                body,
                grid=(2 * T // W_SC,),
                in_specs=[pl.BlockSpec((1, W_SC), lambda j: (0, j))],
                out_specs=[pl.BlockSpec((W_SC, H // HC),
                                        lambda j, _h=hh: (j, _h))],
                core_axis_name=("c", "s"),
                dimension_semantics=(pltpu.PARALLEL,),
            )(i_hbm, o_hbm)

    return k(ys, pos_sc)


def kernel(hidden_states, gate_w, e_w1, e_w2, e_w3, s_w1, s_w2, s_w3, sg_w):
    b, s, n, h = hidden_states.shape
    x = hidden_states.reshape(-1, h)

    logits, xb, pos, w0, w1, gate, meta = pl.pallas_call(
        _router_body,
        grid=(1,),
        in_specs=[
            pl.BlockSpec((T, H), lambda i: (0, 0)),
            pl.BlockSpec((E, H), lambda i: (0, 0)),
            pl.BlockSpec((1, H), lambda i: (0, 0)),
        ],
        out_specs=[
            pl.BlockSpec((T, E), lambda i: (0, 0)),
            pl.BlockSpec((T, H), lambda i: (0, 0)),
            pl.BlockSpec((1, 2 * T), lambda i: (0, 0)),
            pl.BlockSpec((T, 1), lambda i: (0, 0)),
            pl.BlockSpec((T, 1), lambda i: (0, 0)),
            pl.BlockSpec((T, 1), lambda i: (0, 0)),
            pl.BlockSpec((NB, 2), lambda i: (0, 0)),
        ],
        out_shape=[
            jax.ShapeDtypeStruct((T, E), jnp.float32),
            jax.ShapeDtypeStruct((T, H), jnp.bfloat16),
            jax.ShapeDtypeStruct((1, 2 * T), jnp.int32),
            jax.ShapeDtypeStruct((T, 1), jnp.float32),
            jax.ShapeDtypeStruct((T, 1), jnp.float32),
            jax.ShapeDtypeStruct((T, 1), jnp.float32),
            jax.ShapeDtypeStruct((NB, 2), jnp.int32),
        ],
    )(x, gate_w, sg_w)

    pos_sc = pos

    xs = _sc_scatter(x, pos_sc)

    grid_spec = pltpu.PrefetchScalarGridSpec(
        num_scalar_prefetch=1,
        grid=(NB, MOE_INTER // MI2),
        in_specs=[
            pl.BlockSpec((BLK, H), lambda bb, m, s_r: (bb, 0)),
            pl.BlockSpec((1, MI2, H), lambda bb, m, s_r: (s_r[bb, 0], m, 0)),
            pl.BlockSpec((1, MI2, H), lambda bb, m, s_r: (s_r[bb, 0], m, 0)),
            pl.BlockSpec((1, H, MI2), lambda bb, m, s_r: (s_r[bb, 0], 0, m)),
        ],
        out_specs=pl.BlockSpec((BLK, H), lambda bb, m, s_r: (bb, 0)),
        scratch_shapes=[pltpu.VMEM((BLK, H), jnp.float32)],
    )
    ys = pl.pallas_call(
        _grouped_body,
        grid_spec=grid_spec,
        out_shape=jax.ShapeDtypeStruct((NS, H), jnp.float32),
    )(meta, xs, e_w1, e_w3, e_w2)

    yg = _sc_gather(ys, pos_sc)

    out = pl.pallas_call(
        _shared_body,
        grid=(SH_INTER // MI2, T // TMS),
        in_specs=[
            pl.BlockSpec((TMS, H), lambda m, t: (t, 0)),
            pl.BlockSpec((MI2, H), lambda m, t: (m, 0)),
            pl.BlockSpec((MI2, H), lambda m, t: (m, 0)),
            pl.BlockSpec((H, MI2), lambda m, t: (0, m)),
            pl.BlockSpec((TMS, 1), lambda m, t: (t, 0)),
            pl.BlockSpec((TMS, H), lambda m, t: (t, 0)),
            pl.BlockSpec((TMS, H), lambda m, t: (T // TMS + t, 0)),
            pl.BlockSpec((TMS, 1), lambda m, t: (t, 0)),
            pl.BlockSpec((TMS, 1), lambda m, t: (t, 0)),
        ],
        out_specs=pl.BlockSpec((TMS, H), lambda m, t: (t, 0)),
        out_shape=jax.ShapeDtypeStruct((T, H), jnp.float32),
        scratch_shapes=[pltpu.VMEM((T, H), jnp.float32)],
    )(xb, s_w1, s_w3, s_w2, gate, yg, yg, w0, w1)

    return out.reshape(b, s, n, h), logits


# speedup vs baseline: 1.1073x; 1.1073x over previous
import jax
import jax.numpy as jnp
from jax.experimental import pallas as pl
import jax.experimental.pallas.tpu as pltpu
from jax.experimental.pallas import tpu_sc as plsc

E = 8
H = 1024
MOE_INTER = 2048
MI2 = MOE_INTER // 2
SH_INTER = 4096
T = 2048
TM = 512
TMS = 1024
BLK = 512
NB = 16
NS = NB * BLK
W_SC = 128
HC = 4


def _shift_down(v, d):
    return jnp.concatenate([jnp.zeros((d, v.shape[1]), v.dtype), v[:-d, :]], axis=0)


def _shift_right(v, d):
    return jnp.concatenate([jnp.zeros((v.shape[0], d), v.dtype), v[:, :-d]], axis=1)


def _router_body(x_ref, gw_ref, sg_ref, logits_ref, xb_ref, pos_ref, w0_ref,
                 w1_ref, gate_ref, meta_ref):
    x = x_ref[0, :, 0, :]
    xb_ref[...] = x.astype(jnp.bfloat16)
    lg = jax.lax.dot_general(x, gw_ref[...], (((1,), (1,)), ((), ())),
                             preferred_element_type=jnp.float32)
    logits_ref[...] = lg
    p = jax.nn.softmax(lg, axis=-1)
    iota = jax.lax.broadcasted_iota(jnp.int32, p.shape, 1)
    m1 = jnp.max(p, axis=-1, keepdims=True)
    pos1 = jnp.min(jnp.where(p == m1, iota, E), axis=-1, keepdims=True)
    first1 = (iota == pos1).astype(jnp.float32)
    p2 = jnp.where(first1 > 0, -jnp.inf, p)
    m2 = jnp.max(p2, axis=-1, keepdims=True)
    pos2 = jnp.min(jnp.where(p2 == m2, iota, E), axis=-1, keepdims=True)
    first2 = (iota == pos2).astype(jnp.float32)
    w0_ref[...] = m1
    w1_ref[...] = m2
    g = jnp.sum(x * sg_ref[...], axis=-1, keepdims=True)
    gate_ref[...] = jax.nn.sigmoid(g)

    onehot = first1 + first2
    incl = onehot
    d = 1
    while d < T:
        incl = incl + _shift_down(incl, d)
        d *= 2
    excl = incl - onehot
    rank0 = jnp.sum(excl * first1, axis=1, keepdims=True)
    rank1 = jnp.sum(excl * first2, axis=1, keepdims=True)
    cnt = incl[T - 1:T, :]
    pcnt = jnp.ceil(cnt * (1.0 / BLK)) * BLK
    incl8 = pcnt
    d = 1
    while d < E:
        incl8 = incl8 + _shift_right(incl8, d)
        d *= 2
    off = incl8 - pcnt
    slot0 = jnp.sum(off * first1, axis=1, keepdims=True) + rank0
    slot1 = jnp.sum(off * first2, axis=1, keepdims=True) + rank1
    slot01 = jnp.concatenate([slot0, slot1], axis=0)
    pos_ref[...] = slot01.astype(jnp.int32).T

    iota8 = jax.lax.broadcasted_iota(jnp.int32, (1, E), 1).astype(jnp.float32)
    last_e = jnp.max(jnp.where(pcnt > 0, iota8, -1.0))
    biota = jax.lax.broadcasted_iota(jnp.int32, (NB, E), 0).astype(jnp.float32)
    pincl_b = jnp.broadcast_to(incl8, (NB, E))
    ebs = jnp.sum((biota * BLK >= pincl_b).astype(jnp.float32), axis=1,
                  keepdims=True)
    ebs = jnp.minimum(ebs, last_e)
    total = incl8[0:1, E - 1:E]
    bcol = jax.lax.broadcasted_iota(jnp.int32, (NB, 1), 0).astype(jnp.float32)
    act = (bcol * BLK < total).astype(jnp.float32)
    meta_ref[...] = jnp.concatenate([ebs, act], axis=1).astype(jnp.int32)


def _grouped_body(s_ref, xs_ref, w1_ref, w3_ref, w2_ref, ys_ref, acc_ref):
    b = pl.program_id(0)
    m = pl.program_id(1)

    @pl.when(s_ref[b, 1] == 1)
    def _():
        xw = xs_ref[...].astype(jnp.bfloat16)
        w1 = w1_ref[0].astype(jnp.bfloat16)
        w3 = w3_ref[0].astype(jnp.bfloat16)
        w2 = w2_ref[0].astype(jnp.bfloat16)
        a = jax.lax.dot_general(xw, w1, (((1,), (1,)), ((), ())),
                                preferred_element_type=jnp.float32)
        c = jax.lax.dot_general(xw, w3, (((1,), (1,)), ((), ())),
                                preferred_element_type=jnp.float32)
        hm = (a * jax.nn.sigmoid(a) * c).astype(jnp.bfloat16)
        yc = jax.lax.dot_general(hm, w2, (((1,), (1,)), ((), ())),
                                 preferred_element_type=jnp.float32)

        @pl.when(m == 0)
        def _():
            acc_ref[...] = yc

        @pl.when(m != 0)
        def _():
            ys_ref[...] = acc_ref[...] + yc


def _shared_body(xb_ref, s1_ref, s3_ref, s2_ref, gate_ref, o_ref, acc_ref):
    m = pl.program_id(0)
    t = pl.program_id(1)
    xw = xb_ref[...]
    s1 = s1_ref[...].astype(jnp.bfloat16)
    s3 = s3_ref[...].astype(jnp.bfloat16)
    s2 = s2_ref[...].astype(jnp.bfloat16)
    a = jax.lax.dot_general(xw, s1, (((1,), (1,)), ((), ())),
                            preferred_element_type=jnp.float32)
    c = jax.lax.dot_general(xw, s3, (((1,), (1,)), ((), ())),
                            preferred_element_type=jnp.float32)
    hm = (a * jax.nn.sigmoid(a) * c).astype(jnp.bfloat16)
    yc = jax.lax.dot_general(hm, s2, (((1,), (1,)), ((), ())),
                             preferred_element_type=jnp.float32)
    sl = pl.ds(t * TMS, TMS)

    @pl.when(m == 0)
    def _():
        acc_ref[sl, :] = yc

    @pl.when(m != 0)
    def _():
        acc_ref[sl, :] += yc

    o_ref[...] = gate_ref[...] * acc_ref[sl, :]


def _final_body(sh_ref, y0_ref, y1_ref, w0_ref, w1_ref, o_ref):
    o_ref[0, :, 0, :] = (sh_ref[...] + w0_ref[...] * y0_ref[...]
                         + w1_ref[...] * y1_ref[...])


def _sc_scatter(x_f32, pos_sc):
    vector_mesh = plsc.VectorSubcoreMesh(core_axis_name="c",
                                         subcore_axis_name="s")

    @pl.kernel(out_type=jax.ShapeDtypeStruct((NS, H), jnp.float32),
               mesh=vector_mesh)
    def k(x_hbm, i_hbm, o_hbm):
        for hh in range(HC):
            def body(x_vmem, i_vmem, _h=hh):
                pltpu.sync_copy(
                    x_vmem.at[0, :, 0, :],
                    o_hbm.at[i_vmem.at[0], pl.ds(_h * (H // HC), H // HC)])

            pltpu.emit_pipeline(
                body,
                grid=(2 * T // W_SC,),
                in_specs=[
                    pl.BlockSpec((1, W_SC, 1, H // HC),
                                 lambda j, _h=hh: (0, j % (T // W_SC), 0, _h)),
                    pl.BlockSpec((1, W_SC), lambda j: (0, j)),
                ],
                out_specs=[],
                core_axis_name=("c", "s"),
                dimension_semantics=(pltpu.PARALLEL,),
            )(x_hbm, i_hbm)

    return k(x_f32, pos_sc)


def _sc_gather(ys, pos_sc):
    vector_mesh = plsc.VectorSubcoreMesh(core_axis_name="c",
                                         subcore_axis_name="s")

    @pl.kernel(out_type=jax.ShapeDtypeStruct((2 * T, H), jnp.float32),
               mesh=vector_mesh)
    def k(y_hbm, i_hbm, o_hbm):
        for hh in range(HC):
            def body(i_vmem, o_vmem, _h=hh):
                pltpu.sync_copy(
                    y_hbm.at[i_vmem.at[0], pl.ds(_h * (H // HC), H // HC)], o_vmem)

            pltpu.emit_pipeline(
                body,
                grid=(2 * T // W_SC,),
                in_specs=[pl.BlockSpec((1, W_SC), lambda j: (0, j))],
                out_specs=[pl.BlockSpec((W_SC, H // HC),
                                        lambda j, _h=hh: (j, _h))],
                core_axis_name=("c", "s"),
                dimension_semantics=(pltpu.PARALLEL,),
            )(i_hbm, o_hbm)

    return k(ys, pos_sc)


def kernel(hidden_states, gate_w, e_w1, e_w2, e_w3, s_w1, s_w2, s_w3, sg_w):
    b, s, n, h = hidden_states.shape

    logits, xb, pos, w0, w1, gate, meta = pl.pallas_call(
        _router_body,
        grid=(1,),
        in_specs=[
            pl.BlockSpec((1, T, 1, H), lambda i: (0, 0, 0, 0)),
            pl.BlockSpec((E, H), lambda i: (0, 0)),
            pl.BlockSpec((1, H), lambda i: (0, 0)),
        ],
        out_specs=[
            pl.BlockSpec((T, E), lambda i: (0, 0)),
            pl.BlockSpec((T, H), lambda i: (0, 0)),
            pl.BlockSpec((1, 2 * T), lambda i: (0, 0)),
            pl.BlockSpec((T, 1), lambda i: (0, 0)),
            pl.BlockSpec((T, 1), lambda i: (0, 0)),
            pl.BlockSpec((T, 1), lambda i: (0, 0)),
            pl.BlockSpec((NB, 2), lambda i: (0, 0)),
        ],
        out_shape=[
            jax.ShapeDtypeStruct((T, E), jnp.float32),
            jax.ShapeDtypeStruct((T, H), jnp.bfloat16),
            jax.ShapeDtypeStruct((1, 2 * T), jnp.int32),
            jax.ShapeDtypeStruct((T, 1), jnp.float32),
            jax.ShapeDtypeStruct((T, 1), jnp.float32),
            jax.ShapeDtypeStruct((T, 1), jnp.float32),
            jax.ShapeDtypeStruct((NB, 2), jnp.int32),
        ],
    )(hidden_states, gate_w, sg_w)

    pos_sc = pos

    xs = _sc_scatter(hidden_states, pos_sc)

    sh = pl.pallas_call(
        _shared_body,
        grid=(SH_INTER // MI2, T // TMS),
        in_specs=[
            pl.BlockSpec((TMS, H), lambda m, t: (t, 0)),
            pl.BlockSpec((MI2, H), lambda m, t: (m, 0)),
            pl.BlockSpec((MI2, H), lambda m, t: (m, 0)),
            pl.BlockSpec((H, MI2), lambda m, t: (0, m)),
            pl.BlockSpec((TMS, 1), lambda m, t: (t, 0)),
        ],
        out_specs=pl.BlockSpec((TMS, H), lambda m, t: (t, 0)),
        out_shape=jax.ShapeDtypeStruct((T, H), jnp.float32),
        scratch_shapes=[pltpu.VMEM((T, H), jnp.float32)],
    )(xb, s_w1, s_w3, s_w2, gate)

    grid_spec = pltpu.PrefetchScalarGridSpec(
        num_scalar_prefetch=1,
        grid=(NB, MOE_INTER // MI2),
        in_specs=[
            pl.BlockSpec((BLK, H), lambda bb, m, s_r: (bb, 0)),
            pl.BlockSpec((1, MI2, H), lambda bb, m, s_r: (s_r[bb, 0], m, 0)),
            pl.BlockSpec((1, MI2, H), lambda bb, m, s_r: (s_r[bb, 0], m, 0)),
            pl.BlockSpec((1, H, MI2), lambda bb, m, s_r: (s_r[bb, 0], 0, m)),
        ],
        out_specs=pl.BlockSpec((BLK, H), lambda bb, m, s_r: (bb, 0)),
        scratch_shapes=[pltpu.VMEM((BLK, H), jnp.float32)],
    )
    ys = pl.pallas_call(
        _grouped_body,
        grid_spec=grid_spec,
        out_shape=jax.ShapeDtypeStruct((NS, H), jnp.float32),
    )(meta, xs, e_w1, e_w3, e_w2)

    yg = _sc_gather(ys, pos_sc)

    out = pl.pallas_call(
        _final_body,
        grid=(T // TM,),
        in_specs=[
            pl.BlockSpec((TM, H), lambda t: (t, 0)),
            pl.BlockSpec((TM, H), lambda t: (t, 0)),
            pl.BlockSpec((TM, H), lambda t: (T // TM + t, 0)),
            pl.BlockSpec((TM, 1), lambda t: (t, 0)),
            pl.BlockSpec((TM, 1), lambda t: (t, 0)),
        ],
        out_specs=pl.BlockSpec((1, TM, 1, H), lambda t: (0, t, 0, 0)),
        out_shape=jax.ShapeDtypeStruct((1, T, 1, H), jnp.float32),
    )(sh, yg, yg, w0, w1)

    return out, logits


# speedup vs baseline: 1.1082x; 1.0008x over previous
import jax
import jax.numpy as jnp
from jax.experimental import pallas as pl
import jax.experimental.pallas.tpu as pltpu
from jax.experimental.pallas import tpu_sc as plsc

E = 8
H = 1024
MOE_INTER = 2048
MI2 = MOE_INTER // 2
SH_INTER = 4096
T = 2048
TM = 512
TMS = 1024
BLK = 512
NB = 16
NS = NB * BLK
W_SC = 128
HC = 4


def _shift_down(v, d):
    return jnp.concatenate([jnp.zeros((d, v.shape[1]), v.dtype), v[:-d, :]], axis=0)


def _shift_right(v, d):
    return jnp.concatenate([jnp.zeros((v.shape[0], d), v.dtype), v[:, :-d]], axis=1)


def _router_body(x_ref, gw_ref, sg_ref, logits_ref, xb_ref, pos_ref, w0_ref,
                 w1_ref, gate_ref, meta_ref):
    x = x_ref[0, :, 0, :]
    xb_ref[...] = x.astype(jnp.bfloat16)
    lg = jax.lax.dot_general(x, gw_ref[...], (((1,), (1,)), ((), ())),
                             preferred_element_type=jnp.float32)
    logits_ref[...] = lg
    p = jax.nn.softmax(lg, axis=-1)
    iota = jax.lax.broadcasted_iota(jnp.int32, p.shape, 1)
    m1 = jnp.max(p, axis=-1, keepdims=True)
    pos1 = jnp.min(jnp.where(p == m1, iota, E), axis=-1, keepdims=True)
    first1 = (iota == pos1).astype(jnp.float32)
    p2 = jnp.where(first1 > 0, -jnp.inf, p)
    m2 = jnp.max(p2, axis=-1, keepdims=True)
    pos2 = jnp.min(jnp.where(p2 == m2, iota, E), axis=-1, keepdims=True)
    first2 = (iota == pos2).astype(jnp.float32)
    w0_ref[...] = m1
    w1_ref[...] = m2
    g = jnp.sum(x * sg_ref[...], axis=-1, keepdims=True)
    gate_ref[...] = jax.nn.sigmoid(g)

    onehot = first1 + first2
    incl = onehot
    d = 1
    while d < T:
        incl = incl + _shift_down(incl, d)
        d *= 2
    excl = incl - onehot
    rank0 = jnp.sum(excl * first1, axis=1, keepdims=True)
    rank1 = jnp.sum(excl * first2, axis=1, keepdims=True)
    cnt = incl[T - 1:T, :]
    pcnt = jnp.ceil(cnt * (1.0 / BLK)) * BLK
    incl8 = pcnt
    d = 1
    while d < E:
        incl8 = incl8 + _shift_right(incl8, d)
        d *= 2
    off = incl8 - pcnt
    slot0 = jnp.sum(off * first1, axis=1, keepdims=True) + rank0
    slot1 = jnp.sum(off * first2, axis=1, keepdims=True) + rank1
    slot01 = jnp.concatenate([slot0, slot1], axis=0)
    pos_ref[...] = slot01.astype(jnp.int32).T

    iota8 = jax.lax.broadcasted_iota(jnp.int32, (1, E), 1).astype(jnp.float32)
    last_e = jnp.max(jnp.where(pcnt > 0, iota8, -1.0))
    biota = jax.lax.broadcasted_iota(jnp.int32, (NB, E), 0).astype(jnp.float32)
    pincl_b = jnp.broadcast_to(incl8, (NB, E))
    ebs = jnp.sum((biota * BLK >= pincl_b).astype(jnp.float32), axis=1,
                  keepdims=True)
    ebs = jnp.minimum(ebs, last_e)
    total = incl8[0:1, E - 1:E]
    bcol = jax.lax.broadcasted_iota(jnp.int32, (NB, 1), 0).astype(jnp.float32)
    act = (bcol * BLK < total).astype(jnp.float32)
    meta_ref[...] = jnp.concatenate([ebs, act], axis=1).astype(jnp.int32)


def _final_body(sh_ref, y0_ref, y1_ref, w0_ref, w1_ref, o_ref):
    o_ref[0, :, 0, :] = (sh_ref[...] + w0_ref[...] * y0_ref[...]
                         + w1_ref[...] * y1_ref[...])


def _grouped_body(s_ref, xs_ref, w1_ref, w3_ref, w2_ref, ys_ref, acc_ref):
    b = pl.program_id(0)
    m = pl.program_id(1)

    @pl.when(s_ref[b, 1] == 1)
    def _():
        xw = xs_ref[...].astype(jnp.bfloat16)
        w1 = w1_ref[0].astype(jnp.bfloat16)
        w3 = w3_ref[0].astype(jnp.bfloat16)
        w2 = w2_ref[0].astype(jnp.bfloat16)
        a = jax.lax.dot_general(xw, w1, (((1,), (1,)), ((), ())),
                                preferred_element_type=jnp.float32)
        c = jax.lax.dot_general(xw, w3, (((1,), (1,)), ((), ())),
                                preferred_element_type=jnp.float32)
        hm = (a * jax.nn.sigmoid(a) * c).astype(jnp.bfloat16)
        yc = jax.lax.dot_general(hm, w2, (((1,), (1,)), ((), ())),
                                 preferred_element_type=jnp.float32)

        @pl.when(m == 0)
        def _():
            acc_ref[...] = yc

        @pl.when(m != 0)
        def _():
            ys_ref[...] = acc_ref[...] + yc


def _shared_body(xb_ref, s1_ref, s3_ref, s2_ref, gate_ref, o_ref, acc_ref):
    m = pl.program_id(0)
    t = pl.program_id(1)
    xw = xb_ref[...]
    s1 = s1_ref[...].astype(jnp.bfloat16)
    s3 = s3_ref[...].astype(jnp.bfloat16)
    s2 = s2_ref[...].astype(jnp.bfloat16)
    a = jax.lax.dot_general(xw, s1, (((1,), (1,)), ((), ())),
                            preferred_element_type=jnp.float32)
    c = jax.lax.dot_general(xw, s3, (((1,), (1,)), ((), ())),
                            preferred_element_type=jnp.float32)
    hm = (a * jax.nn.sigmoid(a) * c).astype(jnp.bfloat16)
    yc = jax.lax.dot_general(hm, s2, (((1,), (1,)), ((), ())),
                             preferred_element_type=jnp.float32)
    sl = pl.ds(t * TMS, TMS)

    @pl.when(m == 0)
    def _():
        acc_ref[sl, :] = yc

    @pl.when(m != 0)
    def _():
        acc_ref[sl, :] += yc

    o_ref[...] = gate_ref[...] * acc_ref[sl, :]


def _sc_scatter(x_f32, pos_sc):
    vector_mesh = plsc.VectorSubcoreMesh(core_axis_name="c",
                                         subcore_axis_name="s")

    @pl.kernel(out_type=jax.ShapeDtypeStruct((NS, H), jnp.float32),
               mesh=vector_mesh)
    def k(x_hbm, i_hbm, o_hbm):
        for hh in range(HC):
            def body(x_vmem, i_vmem, _h=hh):
                pltpu.sync_copy(
                    x_vmem.at[0, :, 0, :],
                    o_hbm.at[i_vmem.at[0], pl.ds(_h * (H // HC), H // HC)])

            pltpu.emit_pipeline(
                body,
                grid=(2 * T // W_SC,),
                in_specs=[
                    pl.BlockSpec((1, W_SC, 1, H // HC),
                                 lambda j, _h=hh: (0, j % (T // W_SC), 0, _h)),
                    pl.BlockSpec((1, W_SC), lambda j: (0, j)),
                ],
                out_specs=[],
                core_axis_name=("c", "s"),
                dimension_semantics=(pltpu.PARALLEL,),
            )(x_hbm, i_hbm)

    return k(x_f32, pos_sc)


def _sc_gather(ys, pos_sc):
    vector_mesh = plsc.VectorSubcoreMesh(core_axis_name="c",
                                         subcore_axis_name="s")

    @pl.kernel(out_type=jax.ShapeDtypeStruct((2 * T, H), jnp.float32),
               mesh=vector_mesh)
    def k(y_hbm, i_hbm, o_hbm):
        for hh in range(HC):
            def body(i_vmem, o_vmem, _h=hh):
                pltpu.sync_copy(
                    y_hbm.at[i_vmem.at[0], pl.ds(_h * (H // HC), H // HC)], o_vmem)

            pltpu.emit_pipeline(
                body,
                grid=(2 * T // W_SC,),
                in_specs=[pl.BlockSpec((1, W_SC), lambda j: (0, j))],
                out_specs=[pl.BlockSpec((W_SC, H // HC),
                                        lambda j, _h=hh: (j, _h))],
                core_axis_name=("c", "s"),
                dimension_semantics=(pltpu.PARALLEL,),
            )(i_hbm, o_hbm)

    return k(ys, pos_sc)


def kernel(hidden_states, gate_w, e_w1, e_w2, e_w3, s_w1, s_w2, s_w3, sg_w):
    b, s, n, h = hidden_states.shape

    logits, xb, pos, w0, w1, gate, meta = pl.pallas_call(
        _router_body,
        grid=(1,),
        in_specs=[
            pl.BlockSpec((1, T, 1, H), lambda i: (0, 0, 0, 0)),
            pl.BlockSpec((E, H), lambda i: (0, 0)),
            pl.BlockSpec((1, H), lambda i: (0, 0)),
        ],
        out_specs=[
            pl.BlockSpec((T, E), lambda i: (0, 0)),
            pl.BlockSpec((T, H), lambda i: (0, 0)),
            pl.BlockSpec((1, 2 * T), lambda i: (0, 0)),
            pl.BlockSpec((T, 1), lambda i: (0, 0)),
            pl.BlockSpec((T, 1), lambda i: (0, 0)),
            pl.BlockSpec((T, 1), lambda i: (0, 0)),
            pl.BlockSpec((NB, 2), lambda i: (0, 0)),
        ],
        out_shape=[
            jax.ShapeDtypeStruct((T, E), jnp.float32),
            jax.ShapeDtypeStruct((T, H), jnp.bfloat16),
            jax.ShapeDtypeStruct((1, 2 * T), jnp.int32),
            jax.ShapeDtypeStruct((T, 1), jnp.float32),
            jax.ShapeDtypeStruct((T, 1), jnp.float32),
            jax.ShapeDtypeStruct((T, 1), jnp.float32),
            jax.ShapeDtypeStruct((NB, 2), jnp.int32),
        ],
    )(hidden_states, gate_w, sg_w)

    pos_sc = pos

    xs = _sc_scatter(hidden_states, pos_sc)

    sh = pl.pallas_call(
        _shared_body,
        grid=(SH_INTER // MI2, T // TMS),
        in_specs=[
            pl.BlockSpec((TMS, H), lambda m, t: (t, 0)),
            pl.BlockSpec((MI2, H), lambda m, t: (m, 0)),
            pl.BlockSpec((MI2, H), lambda m, t: (m, 0)),
            pl.BlockSpec((H, MI2), lambda m, t: (0, m)),
            pl.BlockSpec((TMS, 1), lambda m, t: (t, 0)),
        ],
        out_specs=pl.BlockSpec((TMS, H), lambda m, t: (t, 0)),
        out_shape=jax.ShapeDtypeStruct((T, H), jnp.float32),
        scratch_shapes=[pltpu.VMEM((T, H), jnp.float32)],
    )(xb, s_w1, s_w3, s_w2, gate)

    grid_spec = pltpu.PrefetchScalarGridSpec(
        num_scalar_prefetch=1,
        grid=(NB, MOE_INTER // MI2),
        in_specs=[
            pl.BlockSpec((BLK, H), lambda bb, m, s_r: (bb, 0)),
            pl.BlockSpec((1, MI2, H), lambda bb, m, s_r: (s_r[bb, 0], m, 0)),
            pl.BlockSpec((1, MI2, H), lambda bb, m, s_r: (s_r[bb, 0], m, 0)),
            pl.BlockSpec((1, H, MI2), lambda bb, m, s_r: (s_r[bb, 0], 0, m)),
        ],
        out_specs=pl.BlockSpec((BLK, H), lambda bb, m, s_r: (bb, 0)),
        scratch_shapes=[pltpu.VMEM((BLK, H), jnp.float32)],
    )
    ys = pl.pallas_call(
        _grouped_body,
        grid_spec=grid_spec,
        out_shape=jax.ShapeDtypeStruct((NS, H), jnp.float32),
    )(meta, xs, e_w1, e_w3, e_w2)

    yg = _sc_gather(ys, pos_sc)

    out = pl.pallas_call(
        _final_body,
        grid=(T // TM,),
        in_specs=[
            pl.BlockSpec((TM, H), lambda t: (t, 0)),
            pl.BlockSpec((TM, H), lambda t: (t, 0)),
            pl.BlockSpec((TM, H), lambda t: (T // TM + t, 0)),
            pl.BlockSpec((TM, 1), lambda t: (t, 0)),
            pl.BlockSpec((TM, 1), lambda t: (t, 0)),
        ],
        out_specs=pl.BlockSpec((1, TM, 1, H), lambda t: (0, t, 0, 0)),
        out_shape=jax.ShapeDtypeStruct((1, T, 1, H), jnp.float32),
    )(sh, yg, yg, w0, w1)

    return out, logits
